# Initial kernel scaffold; baseline (speedup 1.0000x reference)
import jax
import jax.numpy as jnp
from jax.experimental import pallas as pl


def kernel(atom_features, distance, distance_membership_i, distance_membership_j, W_cf, W_df, W_fc, b_cf, b_df):
    raise NotImplementedError("write your pallas kernel here")


# speedup vs baseline: 2.2072x; 2.2072x over previous
import jax
import jax.numpy as jnp
from jax import lax
from jax.experimental import pallas as pl
from jax.experimental.pallas import tpu as pltpu
from jax.experimental.pallas import tpu_sc as plsc

N_ATOMS = 10000
N_PAIRS = 320000
N_EMB = 128
N_DIST = 100
N_HID = 256

NC = 2
NS = 16
NW = NC * NS
PAIRS_PER_W = N_PAIRS // NW
CHUNK = 80
N_CHUNKS = PAIRS_PER_W // CHUNK

ROW_BLK = 2000
PAIR_BLK = 2560


def _prep_body(af_ref, wcf_ref, bcf_ref, bdf_ref, wfc_ref, afh_ref, base_ref):
    af = af_ref[...]
    afh = jnp.dot(af, wcf_ref[...], preferred_element_type=jnp.float32)
    afh = afh + bcf_ref[...]
    afh_ref[...] = afh
    oii = jnp.tanh(jnp.dot(bdf_ref[...] * afh, wfc_ref[...],
                           preferred_element_type=jnp.float32))
    base_ref[...] = af - oii


def _prep(atom_features, W_cf, b_cf2, b_df2, W_fc):
    return pl.pallas_call(
        _prep_body,
        grid=(N_ATOMS // ROW_BLK,),
        in_specs=[
            pl.BlockSpec((ROW_BLK, N_EMB), lambda b: (b, 0)),
            pl.BlockSpec((N_EMB, N_HID), lambda b: (0, 0)),
            pl.BlockSpec((1, N_HID), lambda b: (0, 0)),
            pl.BlockSpec((1, N_HID), lambda b: (0, 0)),
            pl.BlockSpec((N_HID, N_EMB), lambda b: (0, 0)),
        ],
        out_specs=[
            pl.BlockSpec((ROW_BLK, N_HID), lambda b: (b, 0)),
            pl.BlockSpec((ROW_BLK, N_EMB), lambda b: (b, 0)),
        ],
        out_shape=[
            jax.ShapeDtypeStruct((N_ATOMS, N_HID), jnp.float32),
            jax.ShapeDtypeStruct((N_ATOMS, N_EMB), jnp.float32),
        ],
    )(atom_features, W_cf, b_cf2, b_df2, W_fc)


def _gather_body(afh_hbm, j_hbm, out_hbm, idx_v, rows_v, sem):
    wid = lax.axis_index("s") * NC + lax.axis_index("c")
    base = wid * PAIRS_PER_W

    def chunk(c, carry):
        off = pl.multiple_of(base + c * CHUNK, 8)
        pltpu.sync_copy(j_hbm.at[pl.ds(off, CHUNK)], idx_v)
        pltpu.async_copy(afh_hbm.at[idx_v], rows_v, sem).wait()
        pltpu.sync_copy(rows_v, out_hbm.at[pl.ds(off, CHUNK)])
        return carry

    lax.fori_loop(0, N_CHUNKS, chunk, 0)


def _gather(afh, j32):
    return pl.kernel(
        _gather_body,
        out_type=jax.ShapeDtypeStruct((N_PAIRS, N_HID), jnp.float32),
        mesh=plsc.VectorSubcoreMesh(core_axis_name="c", subcore_axis_name="s"),
        scratch_types=[
            pltpu.VMEM((CHUNK,), jnp.int32),
            pltpu.VMEM((CHUNK, N_HID), jnp.float32),
            pltpu.SemaphoreType.DMA,
        ],
    )(afh, j32)


def _pair_body(dist_ref, g_ref, wdf_ref, bdf_ref, wfc_ref, out_ref):
    dh = jnp.dot(dist_ref[...], wdf_ref[...], preferred_element_type=jnp.float32)
    dh = dh + bdf_ref[...]
    t = dh * g_ref[...]
    out_ref[...] = jnp.tanh(jnp.dot(t, wfc_ref[...],
                                    preferred_element_type=jnp.float32))


def _pairs(distance, gathered, W_df, b_df2, W_fc):
    return pl.pallas_call(
        _pair_body,
        grid=(N_PAIRS // PAIR_BLK,),
        in_specs=[
            pl.BlockSpec((PAIR_BLK, N_DIST), lambda b: (b, 0)),
            pl.BlockSpec((PAIR_BLK, N_HID), lambda b: (b, 0)),
            pl.BlockSpec((N_DIST, N_HID), lambda b: (0, 0)),
            pl.BlockSpec((1, N_HID), lambda b: (0, 0)),
            pl.BlockSpec((N_HID, N_EMB), lambda b: (0, 0)),
        ],
        out_specs=pl.BlockSpec((PAIR_BLK, N_EMB), lambda b: (b, 0)),
        out_shape=jax.ShapeDtypeStruct((N_PAIRS, N_EMB), jnp.float32),
    )(distance, gathered, W_df, b_df2, W_fc)


def _scatter_body(rows_hbm, i_hbm, zeros_hbm, out_hbm, idx_v, rows_v, acc_sh):
    c = lax.axis_index("c")
    s = lax.axis_index("s")
    stripe = 640
    tail = N_ATOMS - (NS - 1) * stripe

    @pl.when(s < NS - 1)
    def _():
        pltpu.sync_copy(zeros_hbm.at[pl.ds(s * stripe, stripe)],
                        acc_sh.at[pl.ds(s * stripe, stripe)])

    @pl.when(s == NS - 1)
    def _():
        pltpu.sync_copy(zeros_hbm.at[pl.ds((NS - 1) * stripe, tail)],
                        acc_sh.at[pl.ds((NS - 1) * stripe, tail)])

    plsc.subcore_barrier()

    base = (c * NS + s) * PAIRS_PER_W

    def chunk(k, carry):
        off = pl.multiple_of(base + k * CHUNK, 8)
        pltpu.sync_copy(i_hbm.at[pl.ds(off, CHUNK)], idx_v)
        pltpu.sync_copy(rows_hbm.at[pl.ds(off, CHUNK)], rows_v)
        pltpu.sync_copy(rows_v, acc_sh.at[idx_v], add=True)
        return carry

    lax.fori_loop(0, N_CHUNKS, chunk, 0)
    plsc.subcore_barrier()

    @pl.when(s < NS - 1)
    def _():
        pltpu.sync_copy(acc_sh.at[pl.ds(s * stripe, stripe)],
                        out_hbm.at[c].at[pl.ds(s * stripe, stripe)])

    @pl.when(s == NS - 1)
    def _():
        pltpu.sync_copy(acc_sh.at[pl.ds((NS - 1) * stripe, tail)],
                        out_hbm.at[c].at[pl.ds((NS - 1) * stripe, tail)])


def _scatter(outputs, i32, zeros):
    return pl.kernel(
        _scatter_body,
        out_type=jax.ShapeDtypeStruct((NC, N_ATOMS, N_EMB), jnp.float32),
        mesh=plsc.VectorSubcoreMesh(core_axis_name="c", subcore_axis_name="s"),
        scratch_types=[
            pltpu.VMEM((CHUNK,), jnp.int32),
            pltpu.VMEM((CHUNK, N_EMB), jnp.float32),
            pltpu.VMEM_SHARED((N_ATOMS, N_EMB), jnp.float32),
        ],
    )(outputs, i32, zeros)


def _combine_body(p_ref, base_ref, out_ref):
    out_ref[...] = p_ref[0] + p_ref[1] + base_ref[...]


def _combine(partials, base):
    return pl.pallas_call(
        _combine_body,
        grid=(N_ATOMS // ROW_BLK,),
        in_specs=[
            pl.BlockSpec((NC, ROW_BLK, N_EMB), lambda b: (0, b, 0)),
            pl.BlockSpec((ROW_BLK, N_EMB), lambda b: (b, 0)),
        ],
        out_specs=pl.BlockSpec((ROW_BLK, N_EMB), lambda b: (b, 0)),
        out_shape=jax.ShapeDtypeStruct((N_ATOMS, N_EMB), jnp.float32),
    )(partials, base)


def kernel(atom_features, distance, distance_membership_i,
           distance_membership_j, W_cf, W_df, W_fc, b_cf, b_df):
    i32 = distance_membership_i.astype(jnp.int32)
    j32 = distance_membership_j.astype(jnp.int32)
    b_cf2 = b_cf.reshape(1, N_HID)
    b_df2 = b_df.reshape(1, N_HID)

    afh, base = _prep(atom_features, W_cf, b_cf2, b_df2, W_fc)
    gathered = _gather(afh, j32)
    outputs = _pairs(distance, gathered, W_df, b_df2, W_fc)
    zeros = jnp.zeros((N_ATOMS, N_EMB), jnp.float32)
    partials = _scatter(outputs, i32, zeros)
    return _combine(partials, base)


# speedup vs baseline: 3.5896x; 1.6264x over previous
import jax
import jax.numpy as jnp
from jax import lax
from jax.experimental import pallas as pl
from jax.experimental.pallas import tpu as pltpu
from jax.experimental.pallas import tpu_sc as plsc

N_ATOMS = 10000
N_PAIRS = 320000
N_EMB = 128
N_DIST = 100
N_HID = 256
N_HIDP = N_HID // 2

NC = 2
NS = 16
NW = NC * NS
PAIRS_PER_W = N_PAIRS // NW
CHUNK = 128
N_FULL = PAIRS_PER_W // CHUNK
TAIL = PAIRS_PER_W - N_FULL * CHUNK

ROW_BLK = 2000
PAIR_BLK = 2560


def _prep_body(af_ref, wcf_ref, bcf_ref, bdf_ref, wfc_ref,
               afhp_ref, base_ref):
    af = af_ref[...]
    afh = jnp.dot(af, wcf_ref[...], preferred_element_type=jnp.float32)
    afh = afh + bcf_ref[...]
    bits = lax.bitcast_convert_type(afh, jnp.uint32)
    rnd = bits + jnp.uint32(0x7FFF) + ((bits >> 16) & jnp.uint32(1))
    lo = rnd[:, :N_HIDP] >> 16
    hi = rnd[:, N_HIDP:] & jnp.uint32(0xFFFF0000)
    afhp_ref[...] = lo | hi
    oii = jnp.tanh(jnp.dot(bdf_ref[...] * afh, wfc_ref[...],
                           preferred_element_type=jnp.float32))
    base_ref[...] = af - oii


def _prep(atom_features, W_cf, b_cf2, b_df2, W_fc):
    return pl.pallas_call(
        _prep_body,
        grid=(N_ATOMS // ROW_BLK,),
        in_specs=[
            pl.BlockSpec((ROW_BLK, N_EMB), lambda b: (b, 0)),
            pl.BlockSpec((N_EMB, N_HID), lambda b: (0, 0)),
            pl.BlockSpec((1, N_HID), lambda b: (0, 0)),
            pl.BlockSpec((1, N_HID), lambda b: (0, 0)),
            pl.BlockSpec((N_HID, N_EMB), lambda b: (0, 0)),
        ],
        out_specs=[
            pl.BlockSpec((ROW_BLK, N_HIDP), lambda b: (b, 0)),
            pl.BlockSpec((ROW_BLK, N_EMB), lambda b: (b, 0)),
        ],
        out_shape=[
            jax.ShapeDtypeStruct((N_ATOMS, N_HIDP), jnp.uint32),
            jax.ShapeDtypeStruct((N_ATOMS, N_EMB), jnp.float32),
        ],
    )(atom_features, W_cf, b_cf2, b_df2, W_fc)


def _gather_body(afh_hbm, j_hbm, out_hbm,
                 idx0, idx1, rows0, rows1,
                 si0, si1, sg0, sg1, sw0, sw1):
    wid = lax.axis_index("s") * NC + lax.axis_index("c")
    base = wid * PAIRS_PER_W

    def pair(g, carry):
        o0 = pl.multiple_of(base + (2 * g) * CHUNK, 8)
        o1 = pl.multiple_of(base + (2 * g + 1) * CHUNK, 8)
        ci0 = pltpu.async_copy(j_hbm.at[pl.ds(o0, CHUNK)], idx0, si0)
        ci1 = pltpu.async_copy(j_hbm.at[pl.ds(o1, CHUNK)], idx1, si1)
        ci0.wait()
        cg0 = pltpu.async_copy(afh_hbm.at[idx0], rows0, sg0)
        ci1.wait()
        cg1 = pltpu.async_copy(afh_hbm.at[idx1], rows1, sg1)
        cg0.wait()
        cw0 = pltpu.async_copy(rows0, out_hbm.at[pl.ds(o0, CHUNK)], sw0)
        cg1.wait()
        cw1 = pltpu.async_copy(rows1, out_hbm.at[pl.ds(o1, CHUNK)], sw1)
        cw0.wait()
        cw1.wait()
        return carry

    lax.fori_loop(0, N_FULL // 2, pair, 0)
    ot = pl.multiple_of(base + N_FULL * CHUNK, 8)
    pltpu.sync_copy(j_hbm.at[pl.ds(ot, TAIL)], idx0.at[pl.ds(0, TAIL)])
    pltpu.async_copy(afh_hbm.at[idx0.at[pl.ds(0, TAIL)]],
                     rows0.at[pl.ds(0, TAIL)], sg0).wait()
    pltpu.sync_copy(rows0.at[pl.ds(0, TAIL)], out_hbm.at[pl.ds(ot, TAIL)])


def _gather(afh_packed, j32):
    return pl.kernel(
        _gather_body,
        out_type=jax.ShapeDtypeStruct((N_PAIRS, N_HIDP), jnp.uint32),
        mesh=plsc.VectorSubcoreMesh(core_axis_name="c", subcore_axis_name="s"),
        scratch_types=[
            pltpu.VMEM((CHUNK,), jnp.int32),
            pltpu.VMEM((CHUNK,), jnp.int32),
            pltpu.VMEM((CHUNK, N_HIDP), jnp.uint32),
            pltpu.VMEM((CHUNK, N_HIDP), jnp.uint32),
            pltpu.SemaphoreType.DMA,
            pltpu.SemaphoreType.DMA,
            pltpu.SemaphoreType.DMA,
            pltpu.SemaphoreType.DMA,
            pltpu.SemaphoreType.DMA,
            pltpu.SemaphoreType.DMA,
        ],
    )(afh_packed, j32)


def _pair_body(dist_ref, g_ref, wdf_ref, bdf_ref, wfc_ref, out_ref):
    dh = jnp.dot(dist_ref[...], wdf_ref[...], preferred_element_type=jnp.float32)
    dh = dh + bdf_ref[...]
    gi = g_ref[...]
    g_lo = lax.bitcast_convert_type(gi << 16, jnp.float32)
    g_hi = lax.bitcast_convert_type(gi & jnp.uint32(0xFFFF0000), jnp.float32)
    t_lo = dh[:, :N_HIDP] * g_lo
    t_hi = dh[:, N_HIDP:] * g_hi
    wfc = wfc_ref[...]
    acc = jnp.dot(t_lo, wfc[:N_HIDP, :], preferred_element_type=jnp.float32)
    acc = acc + jnp.dot(t_hi, wfc[N_HIDP:, :], preferred_element_type=jnp.float32)
    out_ref[...] = jnp.tanh(acc)


def _pairs(distance, gathered, W_df, b_df2, W_fc):
    return pl.pallas_call(
        _pair_body,
        grid=(N_PAIRS // PAIR_BLK,),
        in_specs=[
            pl.BlockSpec((PAIR_BLK, N_DIST), lambda b: (b, 0)),
            pl.BlockSpec((PAIR_BLK, N_HIDP), lambda b: (b, 0)),
            pl.BlockSpec((N_DIST, N_HID), lambda b: (0, 0)),
            pl.BlockSpec((1, N_HID), lambda b: (0, 0)),
            pl.BlockSpec((N_HID, N_EMB), lambda b: (0, 0)),
        ],
        out_specs=pl.BlockSpec((PAIR_BLK, N_EMB), lambda b: (b, 0)),
        out_shape=jax.ShapeDtypeStruct((N_PAIRS, N_EMB), jnp.float32),
    )(distance, gathered, W_df, b_df2, W_fc)


def _scatter_body(rows_hbm, i_hbm, zeros_hbm, out_hbm,
                  idx0, idx1, rows0, rows1, acc_sh, s0, s1):
    c = lax.axis_index("c")
    s = lax.axis_index("s")
    stripe = 640
    tail_rows = N_ATOMS - (NS - 1) * stripe

    @pl.when(s < NS - 1)
    def _():
        pltpu.sync_copy(zeros_hbm.at[pl.ds(s * stripe, stripe)],
                        acc_sh.at[pl.ds(s * stripe, stripe)])

    @pl.when(s == NS - 1)
    def _():
        pltpu.sync_copy(zeros_hbm.at[pl.ds((NS - 1) * stripe, tail_rows)],
                        acc_sh.at[pl.ds((NS - 1) * stripe, tail_rows)])

    plsc.subcore_barrier()

    base = (c * NS + s) * PAIRS_PER_W

    def load(ck, idx_v, rows_v, sem):
        off = pl.multiple_of(base + ck * CHUNK, 8)
        pltpu.async_copy(i_hbm.at[pl.ds(off, CHUNK)], idx_v, sem)
        pltpu.async_copy(rows_hbm.at[pl.ds(off, CHUNK)], rows_v, sem)

    def wait2(sem, idx_v, rows_v):
        pltpu.make_async_copy(i_hbm.at[pl.ds(0, CHUNK)], idx_v, sem).wait()
        pltpu.make_async_copy(rows_hbm.at[pl.ds(0, CHUNK)], rows_v, sem).wait()

    load(0, idx0, rows0, s0)

    def loop(g, carry):
        c0 = 2 * g
        c1 = 2 * g + 1
        load(c1, idx1, rows1, s1)
        wait2(s0, idx0, rows0)
        pltpu.sync_copy(rows0, acc_sh.at[idx0], add=True)

        @pl.when(c0 + 2 < N_FULL)
        def _():
            load(c0 + 2, idx0, rows0, s0)

        wait2(s1, idx1, rows1)
        pltpu.sync_copy(rows1, acc_sh.at[idx1], add=True)
        return carry

    lax.fori_loop(0, N_FULL // 2, loop, 0)

    ot = pl.multiple_of(base + N_FULL * CHUNK, 8)
    pltpu.sync_copy(i_hbm.at[pl.ds(ot, TAIL)], idx0.at[pl.ds(0, TAIL)])
    pltpu.sync_copy(rows_hbm.at[pl.ds(ot, TAIL)], rows0.at[pl.ds(0, TAIL)])
    pltpu.sync_copy(rows0.at[pl.ds(0, TAIL)],
                    acc_sh.at[idx0.at[pl.ds(0, TAIL)]], add=True)

    plsc.subcore_barrier()

    @pl.when(s < NS - 1)
    def _():
        pltpu.sync_copy(acc_sh.at[pl.ds(s * stripe, stripe)],
                        out_hbm.at[c].at[pl.ds(s * stripe, stripe)])

    @pl.when(s == NS - 1)
    def _():
        pltpu.sync_copy(acc_sh.at[pl.ds((NS - 1) * stripe, tail_rows)],
                        out_hbm.at[c].at[pl.ds((NS - 1) * stripe, tail_rows)])


def _scatter(outputs, i32, zeros):
    return pl.kernel(
        _scatter_body,
        out_type=jax.ShapeDtypeStruct((NC, N_ATOMS, N_EMB), jnp.float32),
        mesh=plsc.VectorSubcoreMesh(core_axis_name="c", subcore_axis_name="s"),
        scratch_types=[
            pltpu.VMEM((CHUNK,), jnp.int32),
            pltpu.VMEM((CHUNK,), jnp.int32),
            pltpu.VMEM((CHUNK, N_EMB), jnp.float32),
            pltpu.VMEM((CHUNK, N_EMB), jnp.float32),
            pltpu.VMEM_SHARED((N_ATOMS, N_EMB), jnp.float32),
            pltpu.SemaphoreType.DMA,
            pltpu.SemaphoreType.DMA,
        ],
    )(outputs, i32, zeros)


def _combine_body(p_ref, base_ref, out_ref):
    out_ref[...] = p_ref[0] + p_ref[1] + base_ref[...]


def _combine(partials, base):
    return pl.pallas_call(
        _combine_body,
        grid=(N_ATOMS // ROW_BLK,),
        in_specs=[
            pl.BlockSpec((NC, ROW_BLK, N_EMB), lambda b: (0, b, 0)),
            pl.BlockSpec((ROW_BLK, N_EMB), lambda b: (b, 0)),
        ],
        out_specs=pl.BlockSpec((ROW_BLK, N_EMB), lambda b: (b, 0)),
        out_shape=jax.ShapeDtypeStruct((N_ATOMS, N_EMB), jnp.float32),
    )(partials, base)


def kernel(atom_features, distance, distance_membership_i,
           distance_membership_j, W_cf, W_df, W_fc, b_cf, b_df):
    i32 = distance_membership_i.astype(jnp.int32)
    j32 = distance_membership_j.astype(jnp.int32)
    b_cf2 = b_cf.reshape(1, N_HID)
    b_df2 = b_df.reshape(1, N_HID)

    afh_packed, base = _prep(atom_features, W_cf, b_cf2, b_df2, W_fc)
    gathered = _gather(afh_packed, j32)
    outputs = _pairs(distance, gathered, W_df, b_df2, W_fc)
    zeros = jnp.zeros((N_ATOMS, N_EMB), jnp.float32)
    partials = _scatter(outputs, i32, zeros)
    return _combine(partials, base)
